# Initial kernel scaffold; baseline (speedup 1.0000x reference)
import jax
import jax.numpy as jnp
from jax.experimental import pallas as pl


def kernel(x, weight):
    raise NotImplementedError("write your pallas kernel here")


# speedup vs baseline: 1.4825x; 1.4825x over previous
import functools

import jax
import jax.numpy as jnp
from jax import lax
from jax.experimental import pallas as pl
from jax.experimental.pallas import tpu as pltpu
from jax.experimental.pallas import tpu_sc as plsc

_NUM_CORES = 2
_NUM_SUBCORES = 16
_NW = _NUM_CORES * _NUM_SUBCORES


@functools.lru_cache(maxsize=None)
def _make_gather(B: int, V: int, D: int):
    assert B % _NW == 0
    b_per_w = B // _NW
    chunk = 1280
    assert b_per_w % chunk == 0
    n_chunks = b_per_w // chunk

    mesh = plsc.VectorSubcoreMesh(
        core_axis_name="c", subcore_axis_name="s",
        num_cores=_NUM_CORES, num_subcores=_NUM_SUBCORES)

    @functools.partial(
        pl.kernel,
        mesh=mesh,
        compiler_params=pltpu.CompilerParams(use_tc_tiling_on_sc=False),
        out_type=jax.ShapeDtypeStruct((B, D), jnp.float32),
        scratch_types=[
            pltpu.VMEM((b_per_w,), jnp.int32),
            pltpu.VMEM((chunk, D), jnp.float32),
            pltpu.SemaphoreType.DMA,
        ],
    )
    def gather_kernel(idx_hbm, table_hbm, out_hbm, idx_v, rows_v, gsem):
        wid = lax.axis_index("s") * _NUM_CORES + lax.axis_index("c")
        base = wid * b_per_w
        pltpu.sync_copy(idx_hbm.at[pl.ds(base, b_per_w)], idx_v)

        def body(c, carry):
            off = c * chunk
            pltpu.async_copy(
                table_hbm.at[idx_v.at[pl.ds(off, chunk)]],
                rows_v, gsem).wait()
            pltpu.sync_copy(rows_v, out_hbm.at[pl.ds(base + off, chunk)])
            return carry

        lax.fori_loop(0, n_chunks, body, 0)

    return gather_kernel


@jax.jit
def kernel(x, weight):
    bsz, hist = x.shape
    V, D = weight.shape
    flat = x.reshape(bsz * hist).astype(jnp.int32)
    out = _make_gather(bsz * hist, V, D)(flat, weight)
    return out.reshape(bsz, hist, D)


# speedup vs baseline: 1.4995x; 1.0114x over previous
import functools

import jax
import jax.numpy as jnp
from jax import lax
from jax.experimental import pallas as pl
from jax.experimental.pallas import tpu as pltpu
from jax.experimental.pallas import tpu_sc as plsc

_NUM_CORES = 2
_NUM_SUBCORES = 16
_NW = _NUM_CORES * _NUM_SUBCORES

_CHUNK = 640
_NBUF = 4
_LOOKAHEAD = 2


@functools.lru_cache(maxsize=None)
def _make_gather(B: int, V: int, D: int):
    assert B % _NW == 0
    b_per_w = B // _NW
    chunk, nbuf, k = _CHUNK, _NBUF, _LOOKAHEAD
    assert b_per_w % chunk == 0
    n_chunks = b_per_w // chunk
    assert n_chunks % nbuf == 0 and nbuf > k

    mesh = plsc.VectorSubcoreMesh(
        core_axis_name="c", subcore_axis_name="s",
        num_cores=_NUM_CORES, num_subcores=_NUM_SUBCORES)

    @functools.partial(
        pl.kernel,
        mesh=mesh,
        compiler_params=pltpu.CompilerParams(use_tc_tiling_on_sc=False),
        out_type=jax.ShapeDtypeStruct((B, D), jnp.float32),
        scratch_types=[
            pltpu.VMEM((b_per_w,), jnp.int32),
            [pltpu.VMEM((chunk, D), jnp.float32) for _ in range(nbuf)],
            [pltpu.SemaphoreType.DMA for _ in range(nbuf)],
            [pltpu.SemaphoreType.DMA for _ in range(nbuf)],
        ],
    )
    def gather_kernel(idx_hbm, table_hbm, out_hbm, idx_v, rows, gsems, wsems):
        wid = lax.axis_index("s") * _NUM_CORES + lax.axis_index("c")
        base = wid * b_per_w
        pltpu.sync_copy(idx_hbm.at[pl.ds(base, b_per_w)], idx_v)

        def g_desc(c, b):
            return pltpu.make_async_copy(
                table_hbm.at[idx_v.at[pl.ds(c * chunk, chunk)]],
                rows[b], gsems[b])

        def w_desc(c, b):
            return pltpu.make_async_copy(
                rows[b], out_hbm.at[pl.ds(base + c * chunk, chunk)], wsems[b])

        for b in range(k):
            g_desc(b, b).start()

        def outer(o, carry):
            for b in range(nbuf):
                c = o * nbuf + b
                pb = (b + k) % nbuf
                pc = c + k

                @pl.when(pc < n_chunks)
                def _(pc=pc, pb=pb):
                    @pl.when(pc >= nbuf)
                    def _():
                        w_desc(pc - nbuf, pb).wait()
                    g_desc(pc, pb).start()

                g_desc(c, b).wait()
                w_desc(c, b).start()
            return carry

        lax.fori_loop(0, n_chunks // nbuf, outer, 0)
        for b in range(nbuf):
            w_desc(n_chunks - nbuf + b, b).wait()

    return gather_kernel


@jax.jit
def kernel(x, weight):
    bsz, hist = x.shape
    V, D = weight.shape
    flat = x.reshape(bsz * hist).astype(jnp.int32)
    out = _make_gather(bsz * hist, V, D)(flat, weight)
    return out.reshape(bsz, hist, D)
